# baseline (speedup 1.0000x reference)
import functools

import jax
import jax.numpy as jnp
from jax import lax
from jax.experimental import pallas as pl
from jax.experimental.pallas import tpu as pltpu
from jax.experimental.pallas import tpu_sc as plsc

_B, _S, _D = 4, 4096, 2048
_CHUNK = 256
_SPLIT = _D // _CHUNK
_NROWS = _B * _S * _SPLIT
_NW = 32


def _build_sc_call():
    mesh = plsc.VectorSubcoreMesh(core_axis_name="c", subcore_axis_name="s")

    @functools.partial(
        pl.kernel,
        out_type=jax.ShapeDtypeStruct((_NW, _CHUNK), jnp.float32),
        mesh=mesh,
        scratch_types=[
            pltpu.VMEM((16,), jnp.int32),
            pltpu.VMEM((16,), jnp.int32),
            pltpu.VMEM((1, _CHUNK), jnp.float32),
            pltpu.SemaphoreType.DMA,
        ],
    )
    def gather_kernel(table_hbm, off_hbm, out_hbm, off_v, idx_v, row_v, sem):
        wid = lax.axis_index("s") * 2 + lax.axis_index("c")
        pltpu.sync_copy(off_hbm, off_v)
        offv = off_v[...]
        widv = jnp.full((16,), wid, dtype=jnp.int32)
        bv = lax.shift_right_logical(widv, 3)
        cv = lax.bitwise_and(widv, jnp.full((16,), _SPLIT - 1, dtype=jnp.int32))
        rowv = (bv * _S + offv) * _SPLIT + cv
        idx_v[...] = rowv
        pltpu.async_copy(table_hbm.at[idx_v.at[pl.ds(0, 1)]], row_v, sem).wait()
        pltpu.sync_copy(row_v, out_hbm.at[pl.ds(wid, 1)])

    return gather_kernel


_gather = _build_sc_call()


def kernel(x1, x2):
    table = x1.reshape(_NROWS, _CHUNK)
    off16 = jnp.broadcast_to(x2.reshape(()), (16,)).astype(jnp.int32)
    out = _gather(table, off16)
    return out.reshape(_B, 1, _D)


# speedup vs baseline: 7.3104x; 7.3104x over previous
import functools

import jax
import jax.numpy as jnp
from jax import lax
from jax.experimental import pallas as pl
from jax.experimental.pallas import tpu as pltpu
from jax.experimental.pallas import tpu_sc as plsc

_B, _S, _D = 4, 4096, 2048


def _build_sc_call():
    mesh = plsc.VectorSubcoreMesh(core_axis_name="c", subcore_axis_name="s")

    @functools.partial(
        pl.kernel,
        out_type=jax.ShapeDtypeStruct((_B, _D), jnp.float32),
        mesh=mesh,
        scratch_types=[
            pltpu.VMEM((16,), jnp.int32),
            pltpu.VMEM((16,), jnp.int32),
            pltpu.VMEM((1, _D), jnp.float32),
            pltpu.SemaphoreType.DMA,
        ],
    )
    def gather_kernel(table_hbm, off_hbm, out_hbm, off_v, idx_v, row_v, sem):
        wid = lax.axis_index("s") * 2 + lax.axis_index("c")

        @pl.when(wid < _B)
        def _():
            pltpu.sync_copy(off_hbm, off_v)
            widv = jnp.full((16,), wid, dtype=jnp.int32)
            idx_v[...] = widv * _S + off_v[...]
            pltpu.async_copy(table_hbm.at[idx_v.at[pl.ds(0, 1)]], row_v, sem).wait()
            pltpu.sync_copy(row_v, out_hbm.at[pl.ds(wid, 1)])

    return gather_kernel


_gather = _build_sc_call()


def kernel(x1, x2):
    table = x1.reshape(_B * _S, _D)
    off16 = jnp.broadcast_to(x2.reshape(()), (16,)).astype(jnp.int32)
    out = _gather(table, off16)
    return out.reshape(_B, 1, _D)


# speedup vs baseline: 7.9105x; 1.0821x over previous
import functools

import jax
import jax.numpy as jnp
from jax import lax
from jax.experimental import pallas as pl
from jax.experimental.pallas import tpu as pltpu
from jax.experimental.pallas import tpu_sc as plsc

_B, _S, _D = 4, 4096, 2048


def _build_sc_call():
    mesh = plsc.VectorSubcoreMesh(core_axis_name="c", subcore_axis_name="s")

    @functools.partial(
        pl.kernel,
        out_type=jax.ShapeDtypeStruct((_B, 1, _D), jnp.float32),
        mesh=mesh,
        scratch_types=[
            pltpu.VMEM((16,), jnp.int32),
            pltpu.VMEM((1, _D), jnp.float32),
        ],
    )
    def gather_kernel(x1_hbm, off_hbm, out_hbm, off_v, row_v):
        wid = lax.axis_index("s") * 2 + lax.axis_index("c")

        @pl.when(wid < _B)
        def _():
            pltpu.sync_copy(off_hbm, off_v.at[pl.ds(0, 1)])
            off = off_v[...][0]
            pltpu.sync_copy(x1_hbm.at[wid, pl.ds(off, 1)], row_v)
            pltpu.sync_copy(row_v, out_hbm.at[wid])

    return gather_kernel


_gather = _build_sc_call()


def kernel(x1, x2):
    return _gather(x1, x2.reshape((1,)))


# speedup vs baseline: 8.4941x; 1.0738x over previous
import functools

import jax
import jax.numpy as jnp
from jax import lax
from jax.experimental import pallas as pl
from jax.experimental.pallas import tpu as pltpu
from jax.experimental.pallas import tpu_sc as plsc

_B, _S, _D = 4, 4096, 2048


def _build_sc_call():
    mesh = plsc.VectorSubcoreMesh(
        core_axis_name="c", subcore_axis_name="s", num_cores=1
    )

    @functools.partial(
        pl.kernel,
        out_type=jax.ShapeDtypeStruct((_B, 1, _D), jnp.float32),
        mesh=mesh,
        scratch_types=[
            pltpu.VMEM((16,), jnp.int32),
            pltpu.VMEM((1, _D), jnp.float32),
        ],
    )
    def gather_kernel(x1_hbm, off_hbm, out_hbm, off_v, row_v):
        wid = lax.axis_index("s") + lax.axis_index("c")

        @pl.when(wid < _B)
        def _():
            pltpu.sync_copy(off_hbm, off_v.at[pl.ds(0, 1)])
            off = off_v[...][0]
            pltpu.sync_copy(x1_hbm.at[wid, pl.ds(off, 1)], row_v)
            pltpu.sync_copy(row_v, out_hbm.at[wid])

    return gather_kernel


_gather = _build_sc_call()


def kernel(x1, x2):
    return _gather(x1, x2.reshape((1,)))
